# baseline (speedup 1.0000x reference)
import dataclasses
import functools

import jax
import jax.numpy as jnp
from jax import lax
from jax.experimental import pallas as pl
from jax.experimental.pallas import tpu as pltpu
from jax.experimental.pallas import tpu_sc as plsc

_N, _J, _K = 4096, 1024, 16
_R = 256
_LANES = 16
_TC_ROWS = 2560


def _onehot_tc_kernel(x_ref, o_ref):
    xv = x_ref[...]
    idx = jnp.floor(jnp.clip(xv, 0.0, 15.5)).astype(jnp.int32)
    ks = lax.broadcasted_iota(jnp.int32, (_R, _K, _J), 1)
    o_ref[...] = (idx[:, None, :] == ks).astype(jnp.float32)


def _onehot_tc(x, rows, interpret):
    return pl.pallas_call(
        _onehot_tc_kernel,
        grid=(rows // _R,),
        in_specs=[pl.BlockSpec((_R, _J), lambda g: (g, 0))],
        out_specs=pl.BlockSpec((_R, _K, _J), lambda g: (g, 0, 0)),
        out_shape=jax.ShapeDtypeStruct((rows, _K, _J), jnp.float32),
        compiler_params=pltpu.CompilerParams(
            dimension_semantics=("arbitrary",),
        ),
        interpret=interpret,
    )(x)


def _onehot_sc(x, row_start):
    rows = x.shape[0] - row_start
    mesh = plsc.VectorSubcoreMesh(core_axis_name="core",
                                  subcore_axis_name="subcore")
    cp = pltpu.CompilerParams()
    if "needs_layout_passes" in pltpu.CompilerParams.__dataclass_fields__:
        cp = dataclasses.replace(cp, needs_layout_passes=False)

    @pl.kernel(out_type=jax.ShapeDtypeStruct((rows, _K, _J), jnp.float32),
               mesh=mesh, scratch_types=[], compiler_params=cp)
    def sc_kernel(x_hbm, o_hbm):
        def body(x_vmem, o_vmem):
            @pl.loop(0, _J, step=_LANES)
            def _(c):
                xv = x_vmem.at[0, pl.ds(c, _LANES)][...]
                idx = jnp.clip(xv, 0.0, 15.5).astype(jnp.int32)
                for k in range(_K):
                    o_vmem.at[0, k, pl.ds(c, _LANES)][...] = (
                        idx == k).astype(jnp.float32)

        pltpu.emit_pipeline(
            body,
            grid=(rows,),
            in_specs=[pl.BlockSpec((1, _J),
                                   index_map=lambda i: (i + row_start, 0))],
            out_specs=[pl.BlockSpec((1, _K, _J),
                                    index_map=lambda i: (i, 0, 0))],
            core_axis_name=("core", "subcore"),
            dimension_semantics=(pltpu.PARALLEL,),
        )(x_hbm, o_hbm)

    return sc_kernel(x)


@functools.partial(jax.jit, static_argnames=("interpret",))
def kernel(x, interpret=False):
    if interpret:
        return jnp.transpose(_onehot_tc(x, _N, interpret), (0, 2, 1))
    tc_part = _onehot_tc(x, _TC_ROWS, False)
    sc_part = _onehot_sc(x, _TC_ROWS)
    out = jnp.concatenate([tc_part, sc_part], axis=0)
    return jnp.transpose(out, (0, 2, 1))


# speedup vs baseline: 3.1431x; 3.1431x over previous
import functools

import jax
import jax.numpy as jnp
from jax import lax
from jax.experimental import pallas as pl
from jax.experimental.pallas import tpu as pltpu

_N, _J, _K = 4096, 1024, 16
_R = 256


def _onehot_kernel(x_ref, o_ref):
    xv = x_ref[...]
    idx = jnp.floor(jnp.clip(xv, 0.0, 15.5)).astype(jnp.int32)
    ks = lax.broadcasted_iota(jnp.int32, (_R, _K, _J), 1)
    o_ref[...] = (idx[:, None, :] == ks).astype(jnp.float32)


@functools.partial(jax.jit, static_argnames=("interpret",))
def kernel(x, interpret=False):
    out = pl.pallas_call(
        _onehot_kernel,
        grid=(_N // _R,),
        in_specs=[pl.BlockSpec((_R, _J), lambda g: (g, 0))],
        out_specs=pl.BlockSpec((_R, _K, _J), lambda g: (g, 0, 0)),
        out_shape=jax.ShapeDtypeStruct((_N, _K, _J), jnp.float32),
        compiler_params=pltpu.CompilerParams(
            dimension_semantics=("arbitrary",),
        ),
        interpret=interpret,
    )(x)
    return jnp.transpose(out, (0, 2, 1))
